# Initial kernel scaffold; baseline (speedup 1.0000x reference)
import jax
import jax.numpy as jnp
from jax.experimental import pallas as pl


def kernel(x, edge_index, batch, params):
    raise NotImplementedError("write your pallas kernel here")


# speedup vs baseline: 5.0809x; 5.0809x over previous
import functools

import jax
import jax.numpy as jnp
from jax import lax
from jax.experimental import pallas as pl
from jax.experimental.pallas import tpu as pltpu
from jax.experimental.pallas import tpu_sc as plsc

N = 10000
NP = 10240
D_IN = 128
H = 64
G = 64
E = 320000
CH = 128
GRP = 8
TILES = 32
CHUNKS_PER_TILE = 80
EPAD = TILES * CHUNKS_PER_TILE * CH
NGRP = CHUNKS_PER_TILE // GRP
ROWS_PER_TILE = NP // 16
BR = 1024
NB = NP // BR


def _sc_agg_body(u_hbm, src_hbm, dst_hbm, zer_hbm, out_hbm,
                 sidx, didx, rows, agg_sh, sem):
    cid = lax.axis_index("c")
    sid = lax.axis_index("s")
    wid = cid * 16 + sid

    pltpu.sync_copy(zer_hbm, agg_sh.at[pl.ds(sid * ROWS_PER_TILE, ROWS_PER_TILE)])
    plsc.subcore_barrier()

    def group(g, carry):
        base = wid * CHUNKS_PER_TILE + g * GRP
        pltpu.sync_copy(src_hbm.at[pl.ds(base, GRP)], sidx)
        pltpu.sync_copy(dst_hbm.at[pl.ds(base, GRP)], didx)
        cps = [pltpu.async_copy(u_hbm.at[sidx.at[j]], rows.at[j], sem)
               for j in range(GRP)]
        for cp in cps:
            cp.wait()
        for j in range(GRP):
            pltpu.sync_copy(rows.at[j], agg_sh.at[didx.at[j]], add=True)
        return carry

    lax.fori_loop(0, NGRP, group, 0)
    plsc.subcore_barrier()
    pltpu.sync_copy(agg_sh.at[pl.ds(sid * ROWS_PER_TILE, ROWS_PER_TILE)],
                    out_hbm.at[cid, pl.ds(sid * ROWS_PER_TILE, ROWS_PER_TILE)])


@functools.cache
def _get_sc_agg():
    return pl.kernel(
        _sc_agg_body,
        out_type=jax.ShapeDtypeStruct((2, NP, H), jnp.float32),
        mesh=plsc.VectorSubcoreMesh(core_axis_name="c", subcore_axis_name="s"),
        compiler_params=pltpu.CompilerParams(use_tc_tiling_on_sc=False),
        scratch_types=[
            pltpu.VMEM((GRP, CH), jnp.int32),
            pltpu.VMEM((GRP, CH), jnp.int32),
            pltpu.VMEM((GRP, CH, H), jnp.float32),
            pltpu.VMEM_SHARED((NP, H), jnp.float32),
            pltpu.SemaphoreType.DMA,
        ],
    )


def _mm_body(x_ref, w_ref, o_ref):
    o_ref[...] = lax.dot_general(x_ref[...], w_ref[...],
                                 (((1,), (0,)), ((), ())),
                                 preferred_element_type=jnp.float32)


_mm_first = pl.pallas_call(
    _mm_body,
    grid=(NB,),
    in_specs=[pl.BlockSpec((BR, D_IN), lambda i: (i, 0)),
              pl.BlockSpec((D_IN, H), lambda i: (0, 0))],
    out_specs=pl.BlockSpec((BR, H), lambda i: (i, 0)),
    out_shape=jax.ShapeDtypeStruct((NP, H), jnp.float32),
)


def _comb_body(u_ref, p_ref, epsv_ref, b1_ref, w2_ref, b2_ref, w1n_ref, o_ref):
    z = u_ref[...] * epsv_ref[...] + p_ref[0] + p_ref[1] + b1_ref[...]
    h = jnp.maximum(z, 0.0)
    h2 = lax.dot_general(h, w2_ref[...], (((1,), (0,)), ((), ())),
                         preferred_element_type=jnp.float32) + b2_ref[...]
    o_ref[...] = lax.dot_general(h2, w1n_ref[...], (((1,), (0,)), ((), ())),
                                 preferred_element_type=jnp.float32)


_comb = pl.pallas_call(
    _comb_body,
    grid=(NB,),
    in_specs=[pl.BlockSpec((BR, H), lambda i: (i, 0)),
              pl.BlockSpec((2, BR, H), lambda i: (0, i, 0)),
              pl.BlockSpec((1, H), lambda i: (0, 0)),
              pl.BlockSpec((1, H), lambda i: (0, 0)),
              pl.BlockSpec((H, H), lambda i: (0, 0)),
              pl.BlockSpec((1, H), lambda i: (0, 0)),
              pl.BlockSpec((H, H), lambda i: (0, 0))],
    out_specs=pl.BlockSpec((BR, H), lambda i: (i, 0)),
    out_shape=jax.ShapeDtypeStruct((NP, H), jnp.float32),
)


def _final_body(u_ref, p_ref, epsv_ref, b1_ref, w2_ref, b2_ref,
                ow_ref, ob_ref, bat_ref, o_ref):
    i = pl.program_id(0)
    z = u_ref[...] * epsv_ref[...] + p_ref[0] + p_ref[1] + b1_ref[...]
    h = jnp.maximum(z, 0.0)
    h2 = lax.dot_general(h, w2_ref[...], (((1,), (0,)), ((), ())),
                         preferred_element_type=jnp.float32) + b2_ref[...]
    t = lax.dot_general(h2, ow_ref[...], (((1,), (0,)), ((), ())),
                        preferred_element_type=jnp.float32)
    b = bat_ref[0, 0, :]
    onehot = (b[:, None] == lax.broadcasted_iota(jnp.int32, (BR, G), 1))
    contrib = lax.dot_general(onehot.astype(jnp.float32), t,
                              (((0,), (0,)), ((), ())),
                              preferred_element_type=jnp.float32)

    @pl.when(i == 0)
    def _init():
        o_ref[...] = jnp.broadcast_to(ob_ref[...], (G, 1))

    o_ref[...] += contrib


_final = pl.pallas_call(
    _final_body,
    grid=(NB,),
    in_specs=[pl.BlockSpec((BR, H), lambda i: (i, 0)),
              pl.BlockSpec((2, BR, H), lambda i: (0, i, 0)),
              pl.BlockSpec((1, H), lambda i: (0, 0)),
              pl.BlockSpec((1, H), lambda i: (0, 0)),
              pl.BlockSpec((H, H), lambda i: (0, 0)),
              pl.BlockSpec((1, H), lambda i: (0, 0)),
              pl.BlockSpec((H, 1), lambda i: (0, 0)),
              pl.BlockSpec((1, 1), lambda i: (0, 0)),
              pl.BlockSpec((1, 1, BR), lambda i: (i, 0, 0))],
    out_specs=pl.BlockSpec((G, 1), lambda i: (0, 0)),
    out_shape=jax.ShapeDtypeStruct((G, 1), jnp.float32),
)


def kernel(x, edge_index, batch, params):
    layers = params["layers"]
    src = edge_index[0].astype(jnp.int32)
    dst = edge_index[1].astype(jnp.int32)

    pad = EPAD - E
    src_p = jnp.concatenate([src, jnp.zeros((pad,), jnp.int32)]).reshape(EPAD // CH, CH)
    dst_p = jnp.concatenate([dst, jnp.full((pad,), N, jnp.int32)]).reshape(EPAD // CH, CH)

    x_p = jnp.pad(x, ((0, NP - N), (0, 0)))
    bat3 = jnp.pad(batch.astype(jnp.int32), (0, NP - N),
                   constant_values=G).reshape(NB, 1, BR)
    zer = jnp.zeros((ROWS_PER_TILE, H), jnp.float32)

    u = _mm_first(x_p, layers[0]["W1"])
    pred = None
    for i in range(len(layers)):
        lp = layers[i]
        parts = _get_sc_agg()(u, src_p, dst_p, zer)
        epsv = jnp.broadcast_to(1.0 + lp["eps"], (1, H)).astype(jnp.float32)
        b1 = lp["b1"].reshape(1, H)
        b2 = lp["b2"].reshape(1, H)
        if i + 1 < len(layers):
            u = _comb(u, parts, epsv, b1, lp["W2"], b2, layers[i + 1]["W1"])
        else:
            pred = _final(u, parts, epsv, b1, lp["W2"], b2,
                          params["out_W"], params["out_b"].reshape(1, 1), bat3)
    return pred


# speedup vs baseline: 5.4525x; 1.0731x over previous
import functools

import jax
import jax.numpy as jnp
from jax import lax
from jax.experimental import pallas as pl
from jax.experimental.pallas import tpu as pltpu
from jax.experimental.pallas import tpu_sc as plsc

N = 10000
NP = 10240
D_IN = 128
H = 64
G = 64
E = 320000
CH = 128
GRPP = 4
TILES = 32
CHUNKS_PER_TILE = 80
EPAD = TILES * CHUNKS_PER_TILE * CH
NGG = CHUNKS_PER_TILE // GRPP
ROWS_PER_TILE = NP // 16
BR = 1024
NB = NP // BR


def _sc_agg_body(u_hbm, src_hbm, dst_hbm, zer_hbm, out_hbm,
                 sidx, didx, rows, agg_sh, gsema, gsemb, ssema, ssemb):
    cid = lax.axis_index("c")
    sid = lax.axis_index("s")
    wid = cid * 16 + sid
    cbase = wid * CHUNKS_PER_TILE

    pltpu.sync_copy(src_hbm.at[pl.ds(cbase, CHUNKS_PER_TILE)], sidx)
    pltpu.sync_copy(dst_hbm.at[pl.ds(cbase, CHUNKS_PER_TILE)], didx)
    pltpu.sync_copy(zer_hbm, agg_sh.at[pl.ds(sid * ROWS_PER_TILE, ROWS_PER_TILE)])
    plsc.subcore_barrier()

    def gather(g, buf, sem):
        for j in range(GRPP):
            pltpu.async_copy(u_hbm.at[sidx.at[g * GRPP + j]], rows.at[buf, j], sem)

    def scatter(g, buf, sem):
        for j in range(GRPP):
            pltpu.async_copy(rows.at[buf, j], agg_sh.at[didx.at[g * GRPP + j]],
                             sem, add=True)

    def drain(buf, sem):
        for j in range(GRPP):
            pltpu.make_async_copy(u_hbm.at[pl.ds(0, CH)], rows.at[buf, j], sem).wait()

    gather(0, 0, gsema)

    def sup(t, carry):
        g0 = 2 * t
        g1 = 2 * t + 1
        drain(0, gsema)
        scatter(g0, 0, ssema)

        @pl.when(t > 0)
        def _():
            drain(1, ssemb)

        gather(g1, 1, gsemb)
        drain(1, gsemb)
        scatter(g1, 1, ssemb)
        drain(0, ssema)

        @pl.when(t + 1 < NGG // 2)
        def _():
            gather(g0 + 2, 0, gsema)

        return carry

    lax.fori_loop(0, NGG // 2, sup, 0)
    drain(1, ssemb)
    plsc.subcore_barrier()
    pltpu.sync_copy(agg_sh.at[pl.ds(sid * ROWS_PER_TILE, ROWS_PER_TILE)],
                    out_hbm.at[cid, pl.ds(sid * ROWS_PER_TILE, ROWS_PER_TILE)])


@functools.cache
def _get_sc_agg():
    return pl.kernel(
        _sc_agg_body,
        out_type=jax.ShapeDtypeStruct((2, NP, H), jnp.float32),
        mesh=plsc.VectorSubcoreMesh(core_axis_name="c", subcore_axis_name="s"),
        compiler_params=pltpu.CompilerParams(use_tc_tiling_on_sc=False),
        scratch_types=[
            pltpu.VMEM((CHUNKS_PER_TILE, CH), jnp.int32),
            pltpu.VMEM((CHUNKS_PER_TILE, CH), jnp.int32),
            pltpu.VMEM((2, GRPP, CH, H), jnp.float32),
            pltpu.VMEM_SHARED((NP, H), jnp.float32),
            pltpu.SemaphoreType.DMA,
            pltpu.SemaphoreType.DMA,
            pltpu.SemaphoreType.DMA,
            pltpu.SemaphoreType.DMA,
        ],
    )


def _mm_body(x_ref, w_ref, o_ref):
    o_ref[...] = lax.dot_general(x_ref[...], w_ref[...],
                                 (((1,), (0,)), ((), ())),
                                 preferred_element_type=jnp.float32)


_mm_first = pl.pallas_call(
    _mm_body,
    grid=(NB,),
    in_specs=[pl.BlockSpec((BR, D_IN), lambda i: (i, 0)),
              pl.BlockSpec((D_IN, H), lambda i: (0, 0))],
    out_specs=pl.BlockSpec((BR, H), lambda i: (i, 0)),
    out_shape=jax.ShapeDtypeStruct((NP, H), jnp.float32),
)


def _comb_body(u_ref, p_ref, epsv_ref, b1_ref, w2_ref, b2_ref, w1n_ref, o_ref):
    z = u_ref[...] * epsv_ref[...] + p_ref[0] + p_ref[1] + b1_ref[...]
    h = jnp.maximum(z, 0.0)
    h2 = lax.dot_general(h, w2_ref[...], (((1,), (0,)), ((), ())),
                         preferred_element_type=jnp.float32) + b2_ref[...]
    o_ref[...] = lax.dot_general(h2, w1n_ref[...], (((1,), (0,)), ((), ())),
                                 preferred_element_type=jnp.float32)


_comb = pl.pallas_call(
    _comb_body,
    grid=(NB,),
    in_specs=[pl.BlockSpec((BR, H), lambda i: (i, 0)),
              pl.BlockSpec((2, BR, H), lambda i: (0, i, 0)),
              pl.BlockSpec((1, H), lambda i: (0, 0)),
              pl.BlockSpec((1, H), lambda i: (0, 0)),
              pl.BlockSpec((H, H), lambda i: (0, 0)),
              pl.BlockSpec((1, H), lambda i: (0, 0)),
              pl.BlockSpec((H, H), lambda i: (0, 0))],
    out_specs=pl.BlockSpec((BR, H), lambda i: (i, 0)),
    out_shape=jax.ShapeDtypeStruct((NP, H), jnp.float32),
)


def _final_body(u_ref, p_ref, epsv_ref, b1_ref, w2_ref, b2_ref,
                ow_ref, ob_ref, bat_ref, o_ref):
    i = pl.program_id(0)
    z = u_ref[...] * epsv_ref[...] + p_ref[0] + p_ref[1] + b1_ref[...]
    h = jnp.maximum(z, 0.0)
    h2 = lax.dot_general(h, w2_ref[...], (((1,), (0,)), ((), ())),
                         preferred_element_type=jnp.float32) + b2_ref[...]
    t = lax.dot_general(h2, ow_ref[...], (((1,), (0,)), ((), ())),
                        preferred_element_type=jnp.float32)
    b = bat_ref[0, 0, :]
    onehot = (b[:, None] == lax.broadcasted_iota(jnp.int32, (BR, G), 1))
    contrib = lax.dot_general(onehot.astype(jnp.float32), t,
                              (((0,), (0,)), ((), ())),
                              preferred_element_type=jnp.float32)

    @pl.when(i == 0)
    def _init():
        o_ref[...] = jnp.broadcast_to(ob_ref[...], (G, 1))

    o_ref[...] += contrib


_final = pl.pallas_call(
    _final_body,
    grid=(NB,),
    in_specs=[pl.BlockSpec((BR, H), lambda i: (i, 0)),
              pl.BlockSpec((2, BR, H), lambda i: (0, i, 0)),
              pl.BlockSpec((1, H), lambda i: (0, 0)),
              pl.BlockSpec((1, H), lambda i: (0, 0)),
              pl.BlockSpec((H, H), lambda i: (0, 0)),
              pl.BlockSpec((1, H), lambda i: (0, 0)),
              pl.BlockSpec((H, 1), lambda i: (0, 0)),
              pl.BlockSpec((1, 1), lambda i: (0, 0)),
              pl.BlockSpec((1, 1, BR), lambda i: (i, 0, 0))],
    out_specs=pl.BlockSpec((G, 1), lambda i: (0, 0)),
    out_shape=jax.ShapeDtypeStruct((G, 1), jnp.float32),
)


def kernel(x, edge_index, batch, params):
    layers = params["layers"]
    src = edge_index[0].astype(jnp.int32)
    dst = edge_index[1].astype(jnp.int32)

    pad = EPAD - E
    src_p = jnp.concatenate([src, jnp.zeros((pad,), jnp.int32)]).reshape(EPAD // CH, CH)
    dst_p = jnp.concatenate([dst, jnp.full((pad,), N, jnp.int32)]).reshape(EPAD // CH, CH)

    x_p = jnp.pad(x, ((0, NP - N), (0, 0)))
    bat3 = jnp.pad(batch.astype(jnp.int32), (0, NP - N),
                   constant_values=G).reshape(NB, 1, BR)
    zer = jnp.zeros((ROWS_PER_TILE, H), jnp.float32)

    u = _mm_first(x_p, layers[0]["W1"])
    pred = None
    for i in range(len(layers)):
        lp = layers[i]
        parts = _get_sc_agg()(u, src_p, dst_p, zer)
        epsv = jnp.broadcast_to(1.0 + lp["eps"], (1, H)).astype(jnp.float32)
        b1 = lp["b1"].reshape(1, H)
        b2 = lp["b2"].reshape(1, H)
        if i + 1 < len(layers):
            u = _comb(u, parts, epsv, b1, lp["W2"], b2, layers[i + 1]["W1"])
        else:
            pred = _final(u, parts, epsv, b1, lp["W2"], b2,
                          params["out_W"], params["out_b"].reshape(1, 1), bat3)
    return pred
